# Initial kernel scaffold; baseline (speedup 1.0000x reference)
import jax
import jax.numpy as jnp
from jax.experimental import pallas as pl


def kernel(data, neigh_d2, neigh_d1, neigh_d0, neigh_up, parent_d0, parent_d1, parent_up, depth, params):
    raise NotImplementedError("write your pallas kernel here")


# speedup vs baseline: 3.5441x; 3.5441x over previous
import functools

import jax
import jax.numpy as jnp
from jax import lax
from jax.experimental import pallas as pl
from jax.experimental.pallas import tpu as pltpu
from jax.experimental.pallas import tpu_sc as plsc

F32 = jnp.float32
_HI = lax.Precision.HIGHEST
_NC, _NS = 2, 16
_NW = _NC * _NS


def _bn_for(n, c):
    for b in ((1024, 512, 256, 128, 64) if c <= 64 else (256, 128, 64)):
        if n % b == 0:
            return b
    return n


def _gn_in(x, g, bt):
    c = x.shape[-1]
    ng = max(1, min(32, c // 8))
    gs = c // ng
    ci = lax.broadcasted_iota(jnp.int32, (c, ng), 0)
    gi = lax.broadcasted_iota(jnp.int32, (c, ng), 1)
    m = jnp.where(ci // gs == gi, 1.0 / gs, 0.0).astype(F32)
    gi2 = lax.broadcasted_iota(jnp.int32, (ng, c), 0)
    ci2 = lax.broadcasted_iota(jnp.int32, (ng, c), 1)
    e = jnp.where(ci2 // gs == gi2, 1.0, 0.0).astype(F32)
    mu = jnp.dot(jnp.dot(x, m, precision=_HI), e, precision=_HI)
    d = x - mu
    var = jnp.dot(jnp.dot(d * d, m, precision=_HI), e, precision=_HI)
    return d * lax.rsqrt(var + 1e-5) * g + bt


def _mm8(x, w8, gn_params=None):
    n = x.shape[0]
    cin = w8.shape[1]
    cout = w8.shape[2]
    wp = max(cout, 128)
    bn = _bn_for(n, cin)
    wr = w8.transpose(1, 0, 2).reshape(cin, 8 * cout)
    if wp != cout:
        wr = wr.reshape(cin, 8, cout)
        wr = jnp.concatenate(
            [wr, jnp.zeros((cin, 8, wp - cout), F32)], axis=2).reshape(
                cin, 8 * wp)
    ins = [x, wr]
    in_specs = [pl.BlockSpec((bn, x.shape[1]), lambda i: (i, 0)),
                pl.BlockSpec((cin, 8 * wp), lambda i: (0, 0))]
    if gn_params is not None:
        g, bt = gn_params
        ins += [g.reshape(1, cin), bt.reshape(1, cin)]
        in_specs += [pl.BlockSpec((1, cin), lambda i: (0, 0))] * 2

    def body(*refs):
        if gn_params is None:
            x_ref, w_ref, y_ref = refs
            xb = x_ref[...][:, :cin]
        else:
            x_ref, w_ref, g_ref, bt_ref, y_ref = refs
            xb = jax.nn.gelu(_gn_in(x_ref[...][:, :cin],
                                    g_ref[...], bt_ref[...]))
        y = jnp.dot(xb, w_ref[...], precision=_HI)
        for k in range(8):
            y_ref[k] = y[:, k * wp:(k + 1) * wp]

    out = pl.pallas_call(
        body, grid=(n // bn,), in_specs=in_specs,
        out_specs=pl.BlockSpec((8, bn, wp), lambda i: (0, i, 0)),
        out_shape=jax.ShapeDtypeStruct((8, n, wp), F32),
    )(*ins)
    return out.reshape(8 * n, wp)


def _mm(x, w, pad_to=None):
    n, cin = x.shape
    if pad_to is not None and pad_to > w.shape[1]:
        w = jnp.concatenate(
            [w, jnp.zeros((cin, pad_to - w.shape[1]), F32)], axis=1)
    cout = w.shape[1]
    bn = _bn_for(n, cin)

    def body(x_ref, w_ref, o_ref):
        o_ref[...] = jnp.dot(x_ref[...], w_ref[...], precision=_HI)

    return pl.pallas_call(
        body, grid=(n // bn,),
        in_specs=[pl.BlockSpec((bn, cin), lambda i: (i, 0)),
                  pl.BlockSpec((cin, cout), lambda i: (0, 0))],
        out_specs=pl.BlockSpec((bn, cout), lambda i: (i, 0)),
        out_shape=jax.ShapeDtypeStruct((n, cout), F32),
    )(x, w)


def _fin_resblk(x, z, g, bt):
    n, c = x.shape
    bn = _bn_for(n, c)

    wz = z.shape[1]

    def body(x_ref, z_ref, g_ref, bt_ref, o_ref):
        o_ref[...] = jax.nn.gelu(
            x_ref[...] + _gn_in(z_ref[...][:, :c], g_ref[...], bt_ref[...]))

    return pl.pallas_call(
        body, grid=(n // bn,),
        in_specs=[pl.BlockSpec((bn, c), lambda i: (i, 0)),
                  pl.BlockSpec((bn, wz), lambda i: (i, 0)),
                  pl.BlockSpec((1, c), lambda i: (0, 0)),
                  pl.BlockSpec((1, c), lambda i: (0, 0))],
        out_specs=pl.BlockSpec((bn, c), lambda i: (i, 0)),
        out_shape=jax.ShapeDtypeStruct((n, c), F32),
    )(x, z, g.reshape(1, c), bt.reshape(1, c))


def _up_fin(gath, b, g, bt, skip=None):
    n, wg = gath.shape
    c = g.shape[0]
    bn = _bn_for(n, c)
    ins = [gath, b.reshape(1, c), g.reshape(1, c), bt.reshape(1, c)]
    in_specs = [pl.BlockSpec((bn, wg), lambda i: (i, 0)),
                pl.BlockSpec((1, c), lambda i: (0, 0)),
                pl.BlockSpec((1, c), lambda i: (0, 0)),
                pl.BlockSpec((1, c), lambda i: (0, 0))]
    if skip is not None:
        ins.append(skip)
        in_specs.append(pl.BlockSpec((bn, c), lambda i: (i, 0)))

    def body(*refs):
        if skip is None:
            x_ref, b_ref, g_ref, bt_ref, o_ref = refs
            sk = 0.0
        else:
            x_ref, b_ref, g_ref, bt_ref, s_ref, o_ref = refs
            sk = s_ref[...]
        h = jax.nn.gelu(_gn_in(x_ref[...][:, :c] + b_ref[...],
                               g_ref[...], bt_ref[...]))
        o_ref[...] = h + sk

    return pl.pallas_call(
        body, grid=(n // bn,), in_specs=in_specs,
        out_specs=pl.BlockSpec((bn, c), lambda i: (i, 0)),
        out_shape=jax.ShapeDtypeStruct((n, c), F32),
    )(*ins)


def _cumsum_rows(t):
    n, c = t.shape
    bn = _bn_for(n, c)

    def body(t_ref, o_ref, carry):
        i = pl.program_id(0)

        @pl.when(i == 0)
        def _():
            carry[...] = jnp.zeros_like(carry)

        x = t_ref[...]
        sh = 1
        while sh < bn:
            r = pltpu.roll(x, sh, 0)
            mask = lax.broadcasted_iota(jnp.int32, (bn, c), 0) >= sh
            x = x + jnp.where(mask, r, 0.0)
            sh *= 2
        x = x + carry[...]
        o_ref[...] = x
        carry[...] = x[bn - 1:bn, :]

    return pl.pallas_call(
        body, grid=(n // bn,),
        in_specs=[pl.BlockSpec((bn, c), lambda i: (i, 0))],
        out_specs=pl.BlockSpec((bn, c), lambda i: (i, 0)),
        out_shape=jax.ShapeDtypeStruct((n, c), F32),
        scratch_shapes=[pltpu.VMEM((1, c), F32)],
    )(t)


def _down_fin(g1, g2, meta, b, g, bt):
    p, wg = g1.shape
    c = g.shape[0]
    bp = _bn_for(p, c)

    def body(g1_ref, g2_ref, m_ref, b_ref, g_ref, bt_ref, o_ref):
        starts = m_ref[:, 0:1]
        ends = m_ref[:, 1:2]
        s = (jnp.where(ends > 0, g1_ref[...][:, :c], 0.0)
             - jnp.where(starts > 0, g2_ref[...][:, :c], 0.0))
        cnt = jnp.maximum((ends - starts).astype(F32), 1.0)
        h = s / cnt + b_ref[...]
        o_ref[...] = jax.nn.gelu(_gn_in(h, g_ref[...], bt_ref[...]))

    return pl.pallas_call(
        body, grid=(p // bp,),
        in_specs=[pl.BlockSpec((bp, wg), lambda i: (i, 0)),
                  pl.BlockSpec((bp, wg), lambda i: (i, 0)),
                  pl.BlockSpec((bp, 16), lambda i: (i, 0)),
                  pl.BlockSpec((1, c), lambda i: (0, 0)),
                  pl.BlockSpec((1, c), lambda i: (0, 0)),
                  pl.BlockSpec((1, c), lambda i: (0, 0))],
        out_specs=pl.BlockSpec((bp, c), lambda i: (i, 0)),
        out_shape=jax.ShapeDtypeStruct((p, c), F32),
    )(g1, g2, meta, b.reshape(1, c), g.reshape(1, c), bt.reshape(1, c))


def _down(x, parent, p, prm):
    t = _mm(x, prm['W'], pad_to=128)
    cs = _cumsum_rows(t)
    pr = jnp.arange(p, dtype=jnp.int32)
    starts = jnp.searchsorted(parent, pr).astype(jnp.int32)
    ends = jnp.searchsorted(parent, pr + 1).astype(jnp.int32)
    g1 = _sc_gather(cs, jnp.maximum(ends - 1, 0))
    g2 = _sc_gather(cs, jnp.maximum(starts - 1, 0))
    meta = jnp.zeros((p, 16), jnp.int32).at[:, 0].set(starts).at[:, 1].set(ends)
    return _down_fin(g1, g2, meta, prm['b'], prm['g'], prm['bt'])


def _pred(x, p):
    n, cin = x.shape
    cmid = p['W1'].shape[1]
    cout = p['W2'].shape[1]
    bn = _bn_for(n, cin)

    def body(x_ref, w1_ref, b1_ref, g_ref, bt_ref, w2_ref, b2_ref, o_ref):
        h = jnp.dot(x_ref[...], w1_ref[...], precision=_HI) + b1_ref[...]
        h = jax.nn.gelu(_gn_in(h, g_ref[...], bt_ref[...]))
        o_ref[...] = jnp.dot(h, w2_ref[...], precision=_HI) + b2_ref[...]

    return pl.pallas_call(
        body, grid=(n // bn,),
        in_specs=[pl.BlockSpec((bn, cin), lambda i: (i, 0)),
                  pl.BlockSpec((cin, cmid), lambda i: (0, 0)),
                  pl.BlockSpec((1, cmid), lambda i: (0, 0)),
                  pl.BlockSpec((1, cmid), lambda i: (0, 0)),
                  pl.BlockSpec((1, cmid), lambda i: (0, 0)),
                  pl.BlockSpec((cmid, cout), lambda i: (0, 0)),
                  pl.BlockSpec((1, cout), lambda i: (0, 0))],
        out_specs=pl.BlockSpec((bn, cout), lambda i: (i, 0)),
        out_shape=jax.ShapeDtypeStruct((n, cout), F32),
    )(x, p['W1'], p['b1'].reshape(1, cmid), p['g'].reshape(1, cmid),
      p['bt'].reshape(1, cmid), p['W2'], p['b2'].reshape(1, cout))


def _sc_mesh():
    return plsc.VectorSubcoreMesh(core_axis_name="c", subcore_axis_name="s")


def _sc_gather_sum(y2d, ntoff, n, cout):
    cout = y2d.shape[1]
    r = 64 if cout <= 128 else 32
    nch = n // r
    nloop = (nch + _NW - 1) // _NW
    nl = cout // 16
    ntf = ntoff.reshape(7 * n)

    def body(y_hbm, nt_hbm, out_hbm, buf, i0, i1, i2, i3, i4, i5, i6,
             sem, isem):
        idxbs = [i0, i1, i2, i3, i4, i5, i6]
        wid = lax.axis_index("s") * _NC + lax.axis_index("c")

        def step(it, carry):
            ch = wid + it * _NW

            @pl.when(ch < nch)
            def _():
                base = ch * r
                icps = [pltpu.async_copy(nt_hbm.at[pl.ds(j * n + base, r)],
                                         idxbs[j], isem) for j in range(7)]
                scp = pltpu.async_copy(y_hbm.at[pl.ds(base, r)], buf.at[0], sem)
                for cp in icps:
                    cp.wait()
                gcps = [pltpu.async_copy(y_hbm.at[idxbs[j]], buf.at[j + 1], sem)
                        for j in range(7)]
                scp.wait()
                for cp in gcps:
                    cp.wait()

                def rbody(rr, c0):
                    def lbody(ll, c1):
                        off = ll * 16
                        v = buf[0, rr, pl.ds(off, 16)]
                        for j in range(1, 8):
                            v = v + buf[j, rr, pl.ds(off, 16)]
                        buf[0, rr, pl.ds(off, 16)] = v
                        return c1
                    return lax.fori_loop(0, nl, lbody, c0)

                lax.fori_loop(0, r, rbody, 0)
                pltpu.sync_copy(buf.at[0], out_hbm.at[pl.ds(base, r)])
            return carry

        lax.fori_loop(0, nloop, step, 0)

    f = pl.kernel(
        body, out_type=jax.ShapeDtypeStruct((n, cout), F32), mesh=_sc_mesh(),
        scratch_types=[pltpu.VMEM((8, r, cout), F32)]
                      + [pltpu.VMEM((r,), jnp.int32) for _ in range(7)]
                      + [pltpu.SemaphoreType.DMA, pltpu.SemaphoreType.DMA])
    return f(y2d, ntf)


def _sc_gather(tab, idx):
    nrows, cout = tab.shape
    n = idx.shape[0]
    r = 128
    nch = n // r
    nloop = (nch + _NW - 1) // _NW

    def body(t_hbm, i_hbm, out_hbm, rows, idxv, sem):
        wid = lax.axis_index("s") * _NC + lax.axis_index("c")

        def step(it, carry):
            ch = wid + it * _NW

            @pl.when(ch < nch)
            def _():
                base = ch * r
                pltpu.sync_copy(i_hbm.at[pl.ds(base, r)], idxv)
                pltpu.async_copy(t_hbm.at[idxv], rows, sem).wait()
                pltpu.sync_copy(rows, out_hbm.at[pl.ds(base, r)])
            return carry

        lax.fori_loop(0, nloop, step, 0)

    f = pl.kernel(
        body, out_type=jax.ShapeDtypeStruct((n, cout), F32), mesh=_sc_mesh(),
        scratch_types=[pltpu.VMEM((r, cout), F32),
                       pltpu.VMEM((r,), jnp.int32),
                       pltpu.SemaphoreType.DMA])
    return f(tab, idx)


def _ntoff(neigh):
    n = neigh.shape[0]
    return neigh.T + (jnp.arange(7, dtype=jnp.int32)[:, None] + 1) * n


def _resblk(x, ntoff, p):
    n, c = x.shape
    m = c // 2
    y1 = _mm8(x, p['c1']['W'])
    z1 = _sc_gather_sum(y1, ntoff, n, m)
    y2 = _mm8(z1, p['c2W'], gn_params=(p['c1']['g'], p['c1']['bt']))
    z2 = _sc_gather_sum(y2, ntoff, n, c)
    return _fin_resblk(x, z2, p['c2']['g'], p['c2']['bt'])


def kernel(data, neigh_d2, neigh_d1, neigh_d0, neigh_up, parent_d0,
           parent_d1, parent_up, depth, params):
    del depth
    nt2, nt1, nt0, ntu = (_ntoff(neigh_d2), _ntoff(neigh_d1),
                          _ntoff(neigh_d0), _ntoff(neigh_up))
    nd2, nd1 = neigh_d2.shape[0], neigh_d1.shape[0]

    x = data
    for p in params['enc0']:
        x = _resblk(x, nt0, p)
    skip_d = x

    x1 = _down(x, parent_d0, nd1, params['down0'])
    for p in params['enc1']:
        x1 = _resblk(x1, nt1, p)
    skip_d1 = x1

    x2 = _down(x1, parent_d1, nd2, params['down1'])
    for p in params['enc2']:
        x2 = _resblk(x2, nt2, p)
    out = x2
    for p in params['dec0']:
        out = _resblk(out, nt2, p)

    t = _mm(out, params['up0']['W'])
    gth = _sc_gather(t, parent_d1)
    out = _up_fin(gth, params['up0']['b'], params['up0']['g'],
                  params['up0']['bt'], skip=skip_d1)
    for p in params['dec1']:
        out = _resblk(out, nt1, p)

    t = _mm(out, params['up1']['W'])
    gth = _sc_gather(t, parent_d0)
    out = _up_fin(gth, params['up1']['b'], params['up1']['g'],
                  params['up1']['bt'], skip=skip_d)
    for p in params['dec2']:
        out = _resblk(out, nt0, p)

    s0 = _pred(out, params['regress0'])

    t = _mm(out, params['up_dec']['W'], pad_to=128)
    gth = _sc_gather(t, parent_up)
    y = _up_fin(gth, params['up_dec']['b'], params['up_dec']['g'],
                params['up_dec']['bt'])

    nu = neigh_up.shape[0]
    cd = params['conv_dec']
    yk = _mm8(y, cd['W'])
    z = _sc_gather_sum(yk, ntu, nu, cd['W'].shape[2])
    zero = jnp.zeros((cd['W'].shape[2],), F32)
    y2 = _up_fin(z, zero, cd['g'], cd['bt'])

    s1 = _pred(y2, params['regress1'])
    return (s0, s1)


# speedup vs baseline: 4.5139x; 1.2736x over previous
import functools

import jax
import jax.numpy as jnp
from jax import lax
from jax.experimental import pallas as pl
from jax.experimental.pallas import tpu as pltpu
from jax.experimental.pallas import tpu_sc as plsc

F32 = jnp.float32
_HI = lax.Precision.HIGHEST
_NC, _NS = 2, 16
_NW = _NC * _NS


def _bn_for(n, c):
    for b in ((1024, 512, 256, 128, 64) if c <= 64 else (256, 128, 64)):
        if n % b == 0:
            return b
    return n


def _gn_in(x, g, bt):
    c = x.shape[-1]
    ng = max(1, min(32, c // 8))
    gs = c // ng
    ci = lax.broadcasted_iota(jnp.int32, (c, ng), 0)
    gi = lax.broadcasted_iota(jnp.int32, (c, ng), 1)
    m = jnp.where(ci // gs == gi, 1.0 / gs, 0.0).astype(F32)
    gi2 = lax.broadcasted_iota(jnp.int32, (ng, c), 0)
    ci2 = lax.broadcasted_iota(jnp.int32, (ng, c), 1)
    e = jnp.where(ci2 // gs == gi2, 1.0, 0.0).astype(F32)
    mu = jnp.dot(jnp.dot(x, m, precision=_HI), e, precision=_HI)
    d = x - mu
    var = jnp.dot(jnp.dot(d * d, m, precision=_HI), e, precision=_HI)
    return d * lax.rsqrt(var + 1e-5) * g + bt


def _mm8(x, w8, gn_params=None):
    n = x.shape[0]
    cin = w8.shape[1]
    cout = w8.shape[2]
    wp = max(cout, 128)
    bn = _bn_for(n, cin)
    wr = w8.transpose(1, 0, 2).reshape(cin, 8 * cout)
    if wp != cout:
        wr = wr.reshape(cin, 8, cout)
        wr = jnp.concatenate(
            [wr, jnp.zeros((cin, 8, wp - cout), F32)], axis=2).reshape(
                cin, 8 * wp)
    ins = [x, wr]
    in_specs = [pl.BlockSpec((bn, x.shape[1]), lambda i: (i, 0)),
                pl.BlockSpec((cin, 8 * wp), lambda i: (0, 0))]
    if gn_params is not None:
        g, bt = gn_params
        ins += [g.reshape(1, cin), bt.reshape(1, cin)]
        in_specs += [pl.BlockSpec((1, cin), lambda i: (0, 0))] * 2

    def body(*refs):
        if gn_params is None:
            x_ref, w_ref, y_ref = refs
            xb = x_ref[...][:, :cin]
        else:
            x_ref, w_ref, g_ref, bt_ref, y_ref = refs
            xb = jax.nn.gelu(_gn_in(x_ref[...][:, :cin],
                                    g_ref[...], bt_ref[...]))
        y = jnp.dot(xb, w_ref[...])
        for k in range(8):
            y_ref[k] = y[:, k * wp:(k + 1) * wp]

    out = pl.pallas_call(
        body, grid=(n // bn,), in_specs=in_specs,
        out_specs=pl.BlockSpec((8, bn, wp), lambda i: (0, i, 0)),
        out_shape=jax.ShapeDtypeStruct((8, n, wp), F32),
    )(*ins)
    return out.reshape(8 * n, wp)


def _mm(x, w, pad_to=None):
    n, cin = x.shape
    if pad_to is not None and pad_to > w.shape[1]:
        w = jnp.concatenate(
            [w, jnp.zeros((cin, pad_to - w.shape[1]), F32)], axis=1)
    cout = w.shape[1]
    bn = _bn_for(n, cin)

    def body(x_ref, w_ref, o_ref):
        o_ref[...] = jnp.dot(x_ref[...], w_ref[...])

    return pl.pallas_call(
        body, grid=(n // bn,),
        in_specs=[pl.BlockSpec((bn, cin), lambda i: (i, 0)),
                  pl.BlockSpec((cin, cout), lambda i: (0, 0))],
        out_specs=pl.BlockSpec((bn, cout), lambda i: (i, 0)),
        out_shape=jax.ShapeDtypeStruct((n, cout), F32),
    )(x, w)


def _fin_resblk(x, z, g, bt):
    n, c = x.shape
    bn = _bn_for(n, c)

    wz = z.shape[1]

    def body(x_ref, z_ref, g_ref, bt_ref, o_ref):
        o_ref[...] = jax.nn.gelu(
            x_ref[...] + _gn_in(z_ref[...][:, :c], g_ref[...], bt_ref[...]))

    return pl.pallas_call(
        body, grid=(n // bn,),
        in_specs=[pl.BlockSpec((bn, c), lambda i: (i, 0)),
                  pl.BlockSpec((bn, wz), lambda i: (i, 0)),
                  pl.BlockSpec((1, c), lambda i: (0, 0)),
                  pl.BlockSpec((1, c), lambda i: (0, 0))],
        out_specs=pl.BlockSpec((bn, c), lambda i: (i, 0)),
        out_shape=jax.ShapeDtypeStruct((n, c), F32),
    )(x, z, g.reshape(1, c), bt.reshape(1, c))


def _up_fin(gath, b, g, bt, skip=None):
    n, wg = gath.shape
    c = g.shape[0]
    bn = _bn_for(n, c)
    ins = [gath, b.reshape(1, c), g.reshape(1, c), bt.reshape(1, c)]
    in_specs = [pl.BlockSpec((bn, wg), lambda i: (i, 0)),
                pl.BlockSpec((1, c), lambda i: (0, 0)),
                pl.BlockSpec((1, c), lambda i: (0, 0)),
                pl.BlockSpec((1, c), lambda i: (0, 0))]
    if skip is not None:
        ins.append(skip)
        in_specs.append(pl.BlockSpec((bn, c), lambda i: (i, 0)))

    def body(*refs):
        if skip is None:
            x_ref, b_ref, g_ref, bt_ref, o_ref = refs
            sk = 0.0
        else:
            x_ref, b_ref, g_ref, bt_ref, s_ref, o_ref = refs
            sk = s_ref[...]
        h = jax.nn.gelu(_gn_in(x_ref[...][:, :c] + b_ref[...],
                               g_ref[...], bt_ref[...]))
        o_ref[...] = h + sk

    return pl.pallas_call(
        body, grid=(n // bn,), in_specs=in_specs,
        out_specs=pl.BlockSpec((bn, c), lambda i: (i, 0)),
        out_shape=jax.ShapeDtypeStruct((n, c), F32),
    )(*ins)


def _cumsum_rows(t):
    n, c = t.shape
    bn = _bn_for(n, c)

    def body(t_ref, o_ref, carry):
        i = pl.program_id(0)

        @pl.when(i == 0)
        def _():
            carry[...] = jnp.zeros_like(carry)

        x = t_ref[...]
        sh = 1
        while sh < bn:
            r = pltpu.roll(x, sh, 0)
            mask = lax.broadcasted_iota(jnp.int32, (bn, c), 0) >= sh
            x = x + jnp.where(mask, r, 0.0)
            sh *= 2
        x = x + carry[...]
        o_ref[...] = x
        carry[...] = x[bn - 1:bn, :]

    return pl.pallas_call(
        body, grid=(n // bn,),
        in_specs=[pl.BlockSpec((bn, c), lambda i: (i, 0))],
        out_specs=pl.BlockSpec((bn, c), lambda i: (i, 0)),
        out_shape=jax.ShapeDtypeStruct((n, c), F32),
        scratch_shapes=[pltpu.VMEM((1, c), F32)],
    )(t)


def _down_fin(g1, g2, meta, w, b, g, bt):
    p, cin = g1.shape
    c = g.shape[0]
    bp = _bn_for(p, cin)

    def body(g1_ref, g2_ref, m_ref, w_ref, b_ref, g_ref, bt_ref, o_ref):
        starts = m_ref[:, 0:1]
        ends = m_ref[:, 1:2]
        s = (jnp.where(ends > 0, g1_ref[...], 0.0)
             - jnp.where(starts > 0, g2_ref[...], 0.0))
        cnt = jnp.maximum((ends - starts).astype(F32), 1.0)
        h = jnp.dot(s / cnt, w_ref[...]) + b_ref[...]
        o_ref[...] = jax.nn.gelu(_gn_in(h, g_ref[...], bt_ref[...]))

    return pl.pallas_call(
        body, grid=(p // bp,),
        in_specs=[pl.BlockSpec((bp, cin), lambda i: (i, 0)),
                  pl.BlockSpec((bp, cin), lambda i: (i, 0)),
                  pl.BlockSpec((bp, 16), lambda i: (i, 0)),
                  pl.BlockSpec((cin, c), lambda i: (0, 0)),
                  pl.BlockSpec((1, c), lambda i: (0, 0)),
                  pl.BlockSpec((1, c), lambda i: (0, 0)),
                  pl.BlockSpec((1, c), lambda i: (0, 0))],
        out_specs=pl.BlockSpec((bp, c), lambda i: (i, 0)),
        out_shape=jax.ShapeDtypeStruct((p, c), F32),
    )(g1, g2, meta, w, b.reshape(1, c), g.reshape(1, c), bt.reshape(1, c))


def _down(x, parent, p, prm):
    cs = _cumsum_rows(x)
    pr = jnp.arange(p, dtype=jnp.int32)
    starts = jnp.searchsorted(parent, pr).astype(jnp.int32)
    ends = jnp.searchsorted(parent, pr + 1).astype(jnp.int32)
    g1 = _sc_gather(cs, jnp.maximum(ends - 1, 0))
    g2 = _sc_gather(cs, jnp.maximum(starts - 1, 0))
    meta = jnp.zeros((p, 16), jnp.int32).at[:, 0].set(starts).at[:, 1].set(ends)
    return _down_fin(g1, g2, meta, prm['W'], prm['b'], prm['g'], prm['bt'])


def _pred(x, p):
    n, cin = x.shape
    cmid = p['W1'].shape[1]
    cout = p['W2'].shape[1]
    bn = _bn_for(n, cin)

    def body(x_ref, w1_ref, b1_ref, g_ref, bt_ref, w2_ref, b2_ref, o_ref):
        h = jnp.dot(x_ref[...], w1_ref[...]) + b1_ref[...]
        h = jax.nn.gelu(_gn_in(h, g_ref[...], bt_ref[...]))
        o_ref[...] = jnp.dot(h, w2_ref[...]) + b2_ref[...]

    return pl.pallas_call(
        body, grid=(n // bn,),
        in_specs=[pl.BlockSpec((bn, cin), lambda i: (i, 0)),
                  pl.BlockSpec((cin, cmid), lambda i: (0, 0)),
                  pl.BlockSpec((1, cmid), lambda i: (0, 0)),
                  pl.BlockSpec((1, cmid), lambda i: (0, 0)),
                  pl.BlockSpec((1, cmid), lambda i: (0, 0)),
                  pl.BlockSpec((cmid, cout), lambda i: (0, 0)),
                  pl.BlockSpec((1, cout), lambda i: (0, 0))],
        out_specs=pl.BlockSpec((bn, cout), lambda i: (i, 0)),
        out_shape=jax.ShapeDtypeStruct((n, cout), F32),
    )(x, p['W1'], p['b1'].reshape(1, cmid), p['g'].reshape(1, cmid),
      p['bt'].reshape(1, cmid), p['W2'], p['b2'].reshape(1, cout))


def _sc_mesh():
    return plsc.VectorSubcoreMesh(core_axis_name="c", subcore_axis_name="s")


def _sc_gather_sum(y2d, ntoff, n, cout):
    wp = y2d.shape[1]
    r = 16
    nch = n // r
    nloop = (nch + _NW - 1) // _NW
    nl = cout // 16
    nti = ntoff.reshape(7, nch, r).transpose(1, 0, 2).reshape(nch * 7 * r)

    def body(y_hbm, nt_hbm, out_hbm, g0, g1, s0, s1, i0, i1, sem0, sem1):
        gs, ss, ixs, sems = (g0, g1), (s0, s1), (i0, i1), (sem0, sem1)
        wid = lax.axis_index("s") * _NC + lax.axis_index("c")

        def fire(ch, par):
            base = ch * r
            pltpu.sync_copy(nt_hbm.at[pl.ds(ch * 7 * r, 7 * r)], ixs[par])
            for j in range(7):
                pltpu.make_async_copy(
                    y_hbm.at[ixs[par].at[pl.ds(j * r, r)]],
                    gs[par].at[j], sems[par]).start()
            pltpu.make_async_copy(
                y_hbm.at[pl.ds(base, r)], ss[par], sems[par]).start()

        def drain(ch, par):
            for j in range(7):
                pltpu.make_async_copy(
                    y_hbm.at[ixs[par].at[pl.ds(j * r, r)]],
                    gs[par].at[j], sems[par]).wait()
            pltpu.make_async_copy(
                y_hbm.at[pl.ds(ch * r, r)], ss[par], sems[par]).wait()

        @pl.when(wid < nch)
        def _():
            fire(wid, 0)

        def pair(i2, carry):
            for par in range(2):
                it = i2 * 2 + par
                ch = wid + it * _NW
                chn = ch + _NW

                @pl.when(chn < nch)
                def _():
                    fire(chn, 1 - par)

                @pl.when(ch < nch)
                def _():
                    drain(ch, par)
                    g, s = gs[par], ss[par]

                    def rbody(rr, c0):
                        def lbody(ll, c1):
                            off = ll * 16
                            v = s[rr, pl.ds(off, 16)]
                            for j in range(7):
                                v = v + g[j, rr, pl.ds(off, 16)]
                            s[rr, pl.ds(off, 16)] = v
                            return c1
                        return lax.fori_loop(0, nl, lbody, c0)

                    lax.fori_loop(0, r, rbody, 0)
                    pltpu.sync_copy(s, out_hbm.at[pl.ds(ch * r, r)])
            return carry

        lax.fori_loop(0, (nloop + 1) // 2, pair, 0)

    f = pl.kernel(
        body, out_type=jax.ShapeDtypeStruct((n, wp), F32), mesh=_sc_mesh(),
        scratch_types=[pltpu.VMEM((7, r, wp), F32), pltpu.VMEM((7, r, wp), F32),
                       pltpu.VMEM((r, wp), F32), pltpu.VMEM((r, wp), F32),
                       pltpu.VMEM((7 * r,), jnp.int32),
                       pltpu.VMEM((7 * r,), jnp.int32),
                       pltpu.SemaphoreType.DMA, pltpu.SemaphoreType.DMA])
    return f(y2d, nti)


def _sc_gather(tab, idx):
    nrows, cout = tab.shape
    n = idx.shape[0]
    r = 128
    nch = n // r
    nloop = (nch + _NW - 1) // _NW

    def body(t_hbm, i_hbm, out_hbm, r0, r1, i0, i1, sem0, sem1):
        rows, ixs, sems = (r0, r1), (i0, i1), (sem0, sem1)
        wid = lax.axis_index("s") * _NC + lax.axis_index("c")

        def fire(ch, par):
            pltpu.sync_copy(i_hbm.at[pl.ds(ch * r, r)], ixs[par])
            pltpu.make_async_copy(t_hbm.at[ixs[par]], rows[par],
                                  sems[par]).start()

        @pl.when(wid < nch)
        def _():
            fire(wid, 0)

        def pair(i2, carry):
            for par in range(2):
                it = i2 * 2 + par
                ch = wid + it * _NW
                chn = ch + _NW

                @pl.when(chn < nch)
                def _():
                    fire(chn, 1 - par)

                @pl.when(ch < nch)
                def _():
                    pltpu.make_async_copy(t_hbm.at[ixs[par]], rows[par],
                                          sems[par]).wait()
                    pltpu.sync_copy(rows[par], out_hbm.at[pl.ds(ch * r, r)])
            return carry

        lax.fori_loop(0, (nloop + 1) // 2, pair, 0)

    f = pl.kernel(
        body, out_type=jax.ShapeDtypeStruct((n, cout), F32), mesh=_sc_mesh(),
        scratch_types=[pltpu.VMEM((r, cout), F32), pltpu.VMEM((r, cout), F32),
                       pltpu.VMEM((r,), jnp.int32), pltpu.VMEM((r,), jnp.int32),
                       pltpu.SemaphoreType.DMA, pltpu.SemaphoreType.DMA])
    return f(tab, idx)

def _ntoff(neigh):
    n = neigh.shape[0]
    return neigh.T + (jnp.arange(7, dtype=jnp.int32)[:, None] + 1) * n


def _resblk(x, ntoff, p):
    n, c = x.shape
    m = c // 2
    y1 = _mm8(x, p['c1']['W'])
    z1 = _sc_gather_sum(y1, ntoff, n, m)
    y2 = _mm8(z1, p['c2W'], gn_params=(p['c1']['g'], p['c1']['bt']))
    z2 = _sc_gather_sum(y2, ntoff, n, c)
    return _fin_resblk(x, z2, p['c2']['g'], p['c2']['bt'])


def kernel(data, neigh_d2, neigh_d1, neigh_d0, neigh_up, parent_d0,
           parent_d1, parent_up, depth, params):
    del depth
    nt2, nt1, nt0, ntu = (_ntoff(neigh_d2), _ntoff(neigh_d1),
                          _ntoff(neigh_d0), _ntoff(neigh_up))
    nd2, nd1 = neigh_d2.shape[0], neigh_d1.shape[0]

    x = data
    for p in params['enc0']:
        x = _resblk(x, nt0, p)
    skip_d = x

    x1 = _down(x, parent_d0, nd1, params['down0'])
    for p in params['enc1']:
        x1 = _resblk(x1, nt1, p)
    skip_d1 = x1

    x2 = _down(x1, parent_d1, nd2, params['down1'])
    for p in params['enc2']:
        x2 = _resblk(x2, nt2, p)
    out = x2
    for p in params['dec0']:
        out = _resblk(out, nt2, p)

    t = _mm(out, params['up0']['W'])
    gth = _sc_gather(t, parent_d1)
    out = _up_fin(gth, params['up0']['b'], params['up0']['g'],
                  params['up0']['bt'], skip=skip_d1)
    for p in params['dec1']:
        out = _resblk(out, nt1, p)

    t = _mm(out, params['up1']['W'])
    gth = _sc_gather(t, parent_d0)
    out = _up_fin(gth, params['up1']['b'], params['up1']['g'],
                  params['up1']['bt'], skip=skip_d)
    for p in params['dec2']:
        out = _resblk(out, nt0, p)

    s0 = _pred(out, params['regress0'])

    t = _mm(out, params['up_dec']['W'], pad_to=128)
    gth = _sc_gather(t, parent_up)
    y = _up_fin(gth, params['up_dec']['b'], params['up_dec']['g'],
                params['up_dec']['bt'])

    nu = neigh_up.shape[0]
    cd = params['conv_dec']
    yk = _mm8(y, cd['W'])
    z = _sc_gather_sum(yk, ntu, nu, cd['W'].shape[2])
    zero = jnp.zeros((cd['W'].shape[2],), F32)
    y2 = _up_fin(z, zero, cd['g'], cd['bt'])

    s1 = _pred(y2, params['regress1'])
    return (s0, s1)


# speedup vs baseline: 5.0758x; 1.1245x over previous
import functools

import jax
import jax.numpy as jnp
from jax import lax
from jax.experimental import pallas as pl
from jax.experimental.pallas import tpu as pltpu
from jax.experimental.pallas import tpu_sc as plsc

F32 = jnp.float32
_HI = lax.Precision.HIGHEST
_NC, _NS = 2, 16
_NW = _NC * _NS


def _dotr(a, b):
    return jnp.dot(a.astype(jnp.bfloat16), b.astype(jnp.bfloat16),
                   preferred_element_type=F32)


def _bn_for(n, c):
    for b in ((1024, 512, 256, 128, 64) if c <= 64 else (256, 128, 64)):
        if n % b == 0:
            return b
    return n


def _group8_sum(x):
    bn, c = x.shape
    lane = lax.broadcasted_iota(jnp.int32, (bn, c), 1) % 8
    s = x
    for d in (1, 2, 4):
        a = pltpu.roll(s, c - d, 1)
        b = pltpu.roll(s, 8 - d, 1)
        s = s + jnp.where(lane < 8 - d, a, b)
    return s


def _gn_in(x, g, bt):
    mu = _group8_sum(x) * 0.125
    d = x - mu
    var = _group8_sum(d * d) * 0.125
    return d * lax.rsqrt(var + 1e-5) * g + bt


def _mm8(x, w8, gn_params=None):
    n = x.shape[0]
    cin = w8.shape[1]
    cout = w8.shape[2]
    wp = max(cout, 128)
    bn = _bn_for(n, cin)
    wr = w8.transpose(1, 0, 2).reshape(cin, 8 * cout)
    if wp != cout:
        wr = wr.reshape(cin, 8, cout)
        wr = jnp.concatenate(
            [wr, jnp.zeros((cin, 8, wp - cout), F32)], axis=2).reshape(
                cin, 8 * wp)
    ins = [x, wr]
    in_specs = [pl.BlockSpec((bn, x.shape[1]), lambda i: (i, 0)),
                pl.BlockSpec((cin, 8 * wp), lambda i: (0, 0))]
    if gn_params is not None:
        g, bt = gn_params
        ins += [g.reshape(1, cin), bt.reshape(1, cin)]
        in_specs += [pl.BlockSpec((1, cin), lambda i: (0, 0))] * 2

    def body(*refs):
        if gn_params is None:
            x_ref, w_ref, y_ref = refs
            xb = x_ref[...][:, :cin]
        else:
            x_ref, w_ref, g_ref, bt_ref, y_ref = refs
            xb = jax.nn.gelu(_gn_in(x_ref[...][:, :cin],
                                    g_ref[...], bt_ref[...]))
        y = _dotr(xb, w_ref[...])
        for k in range(8):
            y_ref[k] = y[:, k * wp:(k + 1) * wp]

    out = pl.pallas_call(
        body, grid=(n // bn,), in_specs=in_specs,
        out_specs=pl.BlockSpec((8, bn, wp), lambda i: (0, i, 0)),
        out_shape=jax.ShapeDtypeStruct((8, n, wp), F32),
    )(*ins)
    return out.reshape(8 * n, wp)


def _mm(x, w, pad_to=None):
    n, cin = x.shape
    if pad_to is not None and pad_to > w.shape[1]:
        w = jnp.concatenate(
            [w, jnp.zeros((cin, pad_to - w.shape[1]), F32)], axis=1)
    cout = w.shape[1]
    bn = _bn_for(n, cin)

    def body(x_ref, w_ref, o_ref):
        o_ref[...] = _dotr(x_ref[...], w_ref[...])

    return pl.pallas_call(
        body, grid=(n // bn,),
        in_specs=[pl.BlockSpec((bn, cin), lambda i: (i, 0)),
                  pl.BlockSpec((cin, cout), lambda i: (0, 0))],
        out_specs=pl.BlockSpec((bn, cout), lambda i: (i, 0)),
        out_shape=jax.ShapeDtypeStruct((n, cout), F32),
    )(x, w)


def _fin_resblk(x, z, g, bt):
    n, c = x.shape
    bn = _bn_for(n, c)

    wz = z.shape[1]

    def body(x_ref, z_ref, g_ref, bt_ref, o_ref):
        o_ref[...] = jax.nn.gelu(
            x_ref[...] + _gn_in(z_ref[...][:, :c], g_ref[...], bt_ref[...]))

    return pl.pallas_call(
        body, grid=(n // bn,),
        in_specs=[pl.BlockSpec((bn, c), lambda i: (i, 0)),
                  pl.BlockSpec((bn, wz), lambda i: (i, 0)),
                  pl.BlockSpec((1, c), lambda i: (0, 0)),
                  pl.BlockSpec((1, c), lambda i: (0, 0))],
        out_specs=pl.BlockSpec((bn, c), lambda i: (i, 0)),
        out_shape=jax.ShapeDtypeStruct((n, c), F32),
    )(x, z, g.reshape(1, c), bt.reshape(1, c))


def _up_fin(gath, b, g, bt, skip=None):
    n, wg = gath.shape
    c = g.shape[0]
    bn = _bn_for(n, c)
    ins = [gath, b.reshape(1, c), g.reshape(1, c), bt.reshape(1, c)]
    in_specs = [pl.BlockSpec((bn, wg), lambda i: (i, 0)),
                pl.BlockSpec((1, c), lambda i: (0, 0)),
                pl.BlockSpec((1, c), lambda i: (0, 0)),
                pl.BlockSpec((1, c), lambda i: (0, 0))]
    if skip is not None:
        ins.append(skip)
        in_specs.append(pl.BlockSpec((bn, c), lambda i: (i, 0)))

    def body(*refs):
        if skip is None:
            x_ref, b_ref, g_ref, bt_ref, o_ref = refs
            sk = 0.0
        else:
            x_ref, b_ref, g_ref, bt_ref, s_ref, o_ref = refs
            sk = s_ref[...]
        h = jax.nn.gelu(_gn_in(x_ref[...][:, :c] + b_ref[...],
                               g_ref[...], bt_ref[...]))
        o_ref[...] = h + sk

    return pl.pallas_call(
        body, grid=(n // bn,), in_specs=in_specs,
        out_specs=pl.BlockSpec((bn, c), lambda i: (i, 0)),
        out_shape=jax.ShapeDtypeStruct((n, c), F32),
    )(*ins)


def _cumsum_rows(t):
    n, c = t.shape
    bn = _bn_for(n, c)

    def body(t_ref, o_ref, carry):
        i = pl.program_id(0)

        @pl.when(i == 0)
        def _():
            carry[...] = jnp.zeros_like(carry)

        x = t_ref[...]
        sh = 1
        while sh < bn:
            r = pltpu.roll(x, sh, 0)
            mask = lax.broadcasted_iota(jnp.int32, (bn, c), 0) >= sh
            x = x + jnp.where(mask, r, 0.0)
            sh *= 2
        x = x + carry[...]
        o_ref[...] = x
        carry[...] = x[bn - 1:bn, :]

    return pl.pallas_call(
        body, grid=(n // bn,),
        in_specs=[pl.BlockSpec((bn, c), lambda i: (i, 0))],
        out_specs=pl.BlockSpec((bn, c), lambda i: (i, 0)),
        out_shape=jax.ShapeDtypeStruct((n, c), F32),
        scratch_shapes=[pltpu.VMEM((1, c), F32)],
    )(t)


def _down_fin(g1, g2, meta, w, b, g, bt):
    p, cin = g1.shape
    c = g.shape[0]
    bp = _bn_for(p, cin)

    def body(g1_ref, g2_ref, m_ref, w_ref, b_ref, g_ref, bt_ref, o_ref):
        starts = m_ref[:, 0:1]
        ends = m_ref[:, 1:2]
        s = (jnp.where(ends > 0, g1_ref[...], 0.0)
             - jnp.where(starts > 0, g2_ref[...], 0.0))
        cnt = jnp.maximum((ends - starts).astype(F32), 1.0)
        h = _dotr(s / cnt, w_ref[...]) + b_ref[...]
        o_ref[...] = jax.nn.gelu(_gn_in(h, g_ref[...], bt_ref[...]))

    return pl.pallas_call(
        body, grid=(p // bp,),
        in_specs=[pl.BlockSpec((bp, cin), lambda i: (i, 0)),
                  pl.BlockSpec((bp, cin), lambda i: (i, 0)),
                  pl.BlockSpec((bp, 16), lambda i: (i, 0)),
                  pl.BlockSpec((cin, c), lambda i: (0, 0)),
                  pl.BlockSpec((1, c), lambda i: (0, 0)),
                  pl.BlockSpec((1, c), lambda i: (0, 0)),
                  pl.BlockSpec((1, c), lambda i: (0, 0))],
        out_specs=pl.BlockSpec((bp, c), lambda i: (i, 0)),
        out_shape=jax.ShapeDtypeStruct((p, c), F32),
    )(g1, g2, meta, w, b.reshape(1, c), g.reshape(1, c), bt.reshape(1, c))


def _down(x, parent, p, prm):
    cs = _cumsum_rows(x)
    pr = jnp.arange(p, dtype=jnp.int32)
    starts = jnp.searchsorted(parent, pr).astype(jnp.int32)
    ends = jnp.searchsorted(parent, pr + 1).astype(jnp.int32)
    g1 = _sc_gather(cs, jnp.maximum(ends - 1, 0))
    g2 = _sc_gather(cs, jnp.maximum(starts - 1, 0))
    meta = jnp.zeros((p, 16), jnp.int32).at[:, 0].set(starts).at[:, 1].set(ends)
    return _down_fin(g1, g2, meta, prm['W'], prm['b'], prm['g'], prm['bt'])


def _pred(x, p):
    n, cin = x.shape
    cmid = p['W1'].shape[1]
    cout = p['W2'].shape[1]
    bn = _bn_for(n, cin)

    def body(x_ref, w1_ref, b1_ref, g_ref, bt_ref, w2_ref, b2_ref, o_ref):
        h = _dotr(x_ref[...], w1_ref[...]) + b1_ref[...]
        h = jax.nn.gelu(_gn_in(h, g_ref[...], bt_ref[...]))
        o_ref[...] = _dotr(h, w2_ref[...]) + b2_ref[...]

    return pl.pallas_call(
        body, grid=(n // bn,),
        in_specs=[pl.BlockSpec((bn, cin), lambda i: (i, 0)),
                  pl.BlockSpec((cin, cmid), lambda i: (0, 0)),
                  pl.BlockSpec((1, cmid), lambda i: (0, 0)),
                  pl.BlockSpec((1, cmid), lambda i: (0, 0)),
                  pl.BlockSpec((1, cmid), lambda i: (0, 0)),
                  pl.BlockSpec((cmid, cout), lambda i: (0, 0)),
                  pl.BlockSpec((1, cout), lambda i: (0, 0))],
        out_specs=pl.BlockSpec((bn, cout), lambda i: (i, 0)),
        out_shape=jax.ShapeDtypeStruct((n, cout), F32),
    )(x, p['W1'], p['b1'].reshape(1, cmid), p['g'].reshape(1, cmid),
      p['bt'].reshape(1, cmid), p['W2'], p['b2'].reshape(1, cout))


def _sc_mesh():
    return plsc.VectorSubcoreMesh(core_axis_name="c", subcore_axis_name="s")


def _sc_gather_sum(y2d, ntoff, n, cout):
    wp = y2d.shape[1]
    r = 32 if wp <= 128 else 16
    nch = n // r
    nloop = (nch + _NW - 1) // _NW
    nl = cout // 16
    nti = ntoff.reshape(7, nch, r).transpose(1, 0, 2).reshape(nch * 7 * r)

    def body(y_hbm, nt_hbm, out_hbm, g0, g1, s0, s1, i0, i1, sem0, sem1):
        gs, ss, ixs, sems = (g0, g1), (s0, s1), (i0, i1), (sem0, sem1)
        wid = lax.axis_index("s") * _NC + lax.axis_index("c")

        def fire(ch, par):
            base = ch * r
            pltpu.sync_copy(nt_hbm.at[pl.ds(ch * 7 * r, 7 * r)], ixs[par])
            for j in range(7):
                pltpu.make_async_copy(
                    y_hbm.at[ixs[par].at[pl.ds(j * r, r)]],
                    gs[par].at[j], sems[par]).start()
            pltpu.make_async_copy(
                y_hbm.at[pl.ds(base, r)], ss[par], sems[par]).start()

        def drain(ch, par):
            for j in range(7):
                pltpu.make_async_copy(
                    y_hbm.at[ixs[par].at[pl.ds(j * r, r)]],
                    gs[par].at[j], sems[par]).wait()
            pltpu.make_async_copy(
                y_hbm.at[pl.ds(ch * r, r)], ss[par], sems[par]).wait()

        @pl.when(wid < nch)
        def _():
            fire(wid, 0)

        def pair(i2, carry):
            for par in range(2):
                it = i2 * 2 + par
                ch = wid + it * _NW
                chn = ch + _NW

                @pl.when(chn < nch)
                def _():
                    fire(chn, 1 - par)

                @pl.when(ch < nch)
                def _():
                    drain(ch, par)
                    g, s = gs[par], ss[par]

                    def rbody(rr, c0):
                        def lbody(ll, c1):
                            off = ll * 16
                            v = s[rr, pl.ds(off, 16)]
                            for j in range(7):
                                v = v + g[j, rr, pl.ds(off, 16)]
                            s[rr, pl.ds(off, 16)] = v
                            return c1
                        return lax.fori_loop(0, nl, lbody, c0)

                    lax.fori_loop(0, r, rbody, 0)
                    pltpu.sync_copy(s, out_hbm.at[pl.ds(ch * r, r)])
            return carry

        lax.fori_loop(0, (nloop + 1) // 2, pair, 0)

    f = pl.kernel(
        body, out_type=jax.ShapeDtypeStruct((n, wp), F32), mesh=_sc_mesh(),
        scratch_types=[pltpu.VMEM((7, r, wp), F32), pltpu.VMEM((7, r, wp), F32),
                       pltpu.VMEM((r, wp), F32), pltpu.VMEM((r, wp), F32),
                       pltpu.VMEM((7 * r,), jnp.int32),
                       pltpu.VMEM((7 * r,), jnp.int32),
                       pltpu.SemaphoreType.DMA, pltpu.SemaphoreType.DMA])
    return f(y2d, nti)


def _sc_gather(tab, idx):
    nrows, cout = tab.shape
    n = idx.shape[0]
    r = 128
    nch = n // r
    nloop = (nch + _NW - 1) // _NW

    def body(t_hbm, i_hbm, out_hbm, r0, r1, i0, i1, sem0, sem1):
        rows, ixs, sems = (r0, r1), (i0, i1), (sem0, sem1)
        wid = lax.axis_index("s") * _NC + lax.axis_index("c")

        def fire(ch, par):
            pltpu.sync_copy(i_hbm.at[pl.ds(ch * r, r)], ixs[par])
            pltpu.make_async_copy(t_hbm.at[ixs[par]], rows[par],
                                  sems[par]).start()

        @pl.when(wid < nch)
        def _():
            fire(wid, 0)

        def pair(i2, carry):
            for par in range(2):
                it = i2 * 2 + par
                ch = wid + it * _NW
                chn = ch + _NW

                @pl.when(chn < nch)
                def _():
                    fire(chn, 1 - par)

                @pl.when(ch < nch)
                def _():
                    pltpu.make_async_copy(t_hbm.at[ixs[par]], rows[par],
                                          sems[par]).wait()
                    pltpu.sync_copy(rows[par], out_hbm.at[pl.ds(ch * r, r)])
            return carry

        lax.fori_loop(0, (nloop + 1) // 2, pair, 0)

    f = pl.kernel(
        body, out_type=jax.ShapeDtypeStruct((n, cout), F32), mesh=_sc_mesh(),
        scratch_types=[pltpu.VMEM((r, cout), F32), pltpu.VMEM((r, cout), F32),
                       pltpu.VMEM((r,), jnp.int32), pltpu.VMEM((r,), jnp.int32),
                       pltpu.SemaphoreType.DMA, pltpu.SemaphoreType.DMA])
    return f(tab, idx)

def _ntoff(neigh):
    n = neigh.shape[0]
    return neigh.T + (jnp.arange(7, dtype=jnp.int32)[:, None] + 1) * n


def _resblk(x, ntoff, p):
    n, c = x.shape
    m = c // 2
    y1 = _mm8(x, p['c1']['W'])
    z1 = _sc_gather_sum(y1, ntoff, n, m)
    y2 = _mm8(z1, p['c2W'], gn_params=(p['c1']['g'], p['c1']['bt']))
    z2 = _sc_gather_sum(y2, ntoff, n, c)
    return _fin_resblk(x, z2, p['c2']['g'], p['c2']['bt'])


def kernel(data, neigh_d2, neigh_d1, neigh_d0, neigh_up, parent_d0,
           parent_d1, parent_up, depth, params):
    del depth
    nt2, nt1, nt0, ntu = (_ntoff(neigh_d2), _ntoff(neigh_d1),
                          _ntoff(neigh_d0), _ntoff(neigh_up))
    nd2, nd1 = neigh_d2.shape[0], neigh_d1.shape[0]

    x = data
    for p in params['enc0']:
        x = _resblk(x, nt0, p)
    skip_d = x

    x1 = _down(x, parent_d0, nd1, params['down0'])
    for p in params['enc1']:
        x1 = _resblk(x1, nt1, p)
    skip_d1 = x1

    x2 = _down(x1, parent_d1, nd2, params['down1'])
    for p in params['enc2']:
        x2 = _resblk(x2, nt2, p)
    out = x2
    for p in params['dec0']:
        out = _resblk(out, nt2, p)

    t = _mm(out, params['up0']['W'])
    gth = _sc_gather(t, parent_d1)
    out = _up_fin(gth, params['up0']['b'], params['up0']['g'],
                  params['up0']['bt'], skip=skip_d1)
    for p in params['dec1']:
        out = _resblk(out, nt1, p)

    t = _mm(out, params['up1']['W'])
    gth = _sc_gather(t, parent_d0)
    out = _up_fin(gth, params['up1']['b'], params['up1']['g'],
                  params['up1']['bt'], skip=skip_d)
    for p in params['dec2']:
        out = _resblk(out, nt0, p)

    s0 = _pred(out, params['regress0'])

    t = _mm(out, params['up_dec']['W'], pad_to=128)
    gth = _sc_gather(t, parent_up)
    y = _up_fin(gth, params['up_dec']['b'], params['up_dec']['g'],
                params['up_dec']['bt'])

    nu = neigh_up.shape[0]
    cd = params['conv_dec']
    yk = _mm8(y, cd['W'])
    z = _sc_gather_sum(yk, ntu, nu, cd['W'].shape[2])
    zero = jnp.zeros((cd['W'].shape[2],), F32)
    y2 = _up_fin(z, zero, cd['g'], cd['bt'])

    s1 = _pred(y2, params['regress1'])
    return (s0, s1)
